# baseline (speedup 1.0000x reference)
import functools

import jax
import jax.numpy as jnp
from jax import lax
from jax.experimental import pallas as pl
from jax.experimental.pallas import tpu as pltpu
from jax.experimental.pallas import tpu_sc as plsc

N = 10000
E = 320000
D = 128

NC = 2
NS = 16
NW = NC * NS
EPW = E // NW
C = 128
CHUNKS = 2 * (-(-EPW // (2 * C)))
EPW_PAD = CHUNKS * C
N_PAD = 10112
RPT = N_PAD // NS


def _sc_body(xn_hbm, src_hbm, dst_hbm, out_hbm,
             src_v, dst_v, rows_v, rows_v2, tbl_sh, acc_sh, gsem, gsem2):
    c = lax.axis_index("c")
    s = lax.axis_index("s")
    w = c * NS + s
    pltpu.sync_copy(src_hbm.at[w], src_v)
    pltpu.sync_copy(dst_hbm.at[w], dst_v)
    @pl.when(s < NS - 1)
    def _():
        pltpu.sync_copy(xn_hbm.at[pl.ds(s * RPT, RPT)],
                        tbl_sh.at[pl.ds(s * RPT, RPT)])

    @pl.when(s == NS - 1)
    def _():
        pltpu.sync_copy(xn_hbm.at[pl.ds((NS - 1) * RPT, N - (NS - 1) * RPT)],
                        tbl_sh.at[pl.ds((NS - 1) * RPT, N - (NS - 1) * RPT)])
    zrow = jnp.zeros((32,), jnp.bfloat16)

    def zero_row(j, carry):
        for k in range(D // 32):
            rows_v[j, pl.ds(k * 32, 32)] = zrow
        return carry

    lax.fori_loop(0, C, zero_row, 0)
    for k in range(RPT // C + 1):
        size = C if k < RPT // C else RPT - (RPT // C) * C
        pltpu.sync_copy(rows_v.at[pl.ds(0, size)],
                        acc_sh.at[pl.ds(s * RPT + k * C, size)])
    plsc.subcore_barrier()

    def pair_body(i, carry):
        d0 = pltpu.async_copy(tbl_sh.at[src_v.at[2 * i]], rows_v, gsem)
        d1 = pltpu.async_copy(tbl_sh.at[src_v.at[2 * i + 1]], rows_v2, gsem2)
        d0.wait()
        pltpu.sync_copy(rows_v, acc_sh.at[dst_v.at[2 * i]], add=True)
        d1.wait()
        pltpu.sync_copy(rows_v2, acc_sh.at[dst_v.at[2 * i + 1]], add=True)
        return carry

    lax.fori_loop(0, CHUNKS // 2, pair_body, 0)
    plsc.subcore_barrier()
    pltpu.sync_copy(acc_sh.at[pl.ds(s * RPT, RPT)],
                    out_hbm.at[c, pl.ds(s * RPT, RPT)])


def _prescale_body(x_ref, norm_ref, o_ref):
    o_ref[...] = (x_ref[...] * norm_ref[...]).astype(jnp.bfloat16)


def _apply_body(p_ref, norm_ref, w_ref, b_ref, o_ref):
    acc = ((p_ref[0].astype(jnp.float32) + p_ref[1].astype(jnp.float32))
           * norm_ref[...]).astype(jnp.bfloat16)
    h = lax.dot_general(acc, w_ref[...].astype(jnp.bfloat16),
                        (((1,), (1,)), ((), ())),
                        preferred_element_type=jnp.float32)
    o_ref[...] = jnp.maximum(h + b_ref[...], 0.0)


_ROWS_BLK = 1000


def kernel(x, norm, edge_index, W, b):
    xn = pl.pallas_call(
        _prescale_body,
        out_shape=jax.ShapeDtypeStruct((N, D), jnp.bfloat16),
        grid=(N // _ROWS_BLK,),
        in_specs=[pl.BlockSpec((_ROWS_BLK, D), lambda i: (i, 0)),
                  pl.BlockSpec((_ROWS_BLK, 1), lambda i: (i, 0))],
        out_specs=pl.BlockSpec((_ROWS_BLK, D), lambda i: (i, 0)),
    )(x, norm)

    src = edge_index[0].reshape(NW, EPW)
    dst = edge_index[1].reshape(NW, EPW)
    pad_n = EPW_PAD - EPW
    src_p = jnp.pad(src, ((0, 0), (0, pad_n))).reshape(NW, CHUNKS, C)
    trash = (N + (jnp.arange(pad_n, dtype=jnp.int32) % (N_PAD - N)))
    dst_p = jnp.concatenate(
        [dst, jnp.broadcast_to(trash, (NW, pad_n))], axis=1
    ).reshape(NW, CHUNKS, C)

    mesh = plsc.VectorSubcoreMesh(core_axis_name="c", subcore_axis_name="s")
    parts = pl.kernel(
        _sc_body,
        out_type=jax.ShapeDtypeStruct((NC, N_PAD, D), jnp.bfloat16),
        mesh=mesh,
        compiler_params=pltpu.CompilerParams(use_tc_tiling_on_sc=False),
        scratch_types=[
            pltpu.VMEM((CHUNKS, C), jnp.int32),
            pltpu.VMEM((CHUNKS, C), jnp.int32),
            pltpu.VMEM((C, D), jnp.bfloat16),
            pltpu.VMEM((C, D), jnp.bfloat16),
            pltpu.VMEM_SHARED((N_PAD, D), jnp.bfloat16),
            pltpu.VMEM_SHARED((N_PAD, D), jnp.bfloat16),
            pltpu.SemaphoreType.DMA,
            pltpu.SemaphoreType.DMA,
        ],
    )(xn, src_p, dst_p)

    b2 = b.reshape(1, D)
    h = pl.pallas_call(
        _apply_body,
        out_shape=jax.ShapeDtypeStruct((N, D), jnp.float32),
        grid=(N // _ROWS_BLK,),
        in_specs=[
            pl.BlockSpec((NC, _ROWS_BLK, D), lambda i: (0, i, 0)),
            pl.BlockSpec((_ROWS_BLK, 1), lambda i: (i, 0)),
            pl.BlockSpec((D, D), lambda i: (0, 0)),
            pl.BlockSpec((1, D), lambda i: (0, 0)),
        ],
        out_specs=pl.BlockSpec((_ROWS_BLK, D), lambda i: (i, 0)),
    )(parts, norm, W, b2)
    return h


# speedup vs baseline: 1.1730x; 1.1730x over previous
import functools

import jax
import jax.numpy as jnp
from jax import lax
from jax.experimental import pallas as pl
from jax.experimental.pallas import tpu as pltpu
from jax.experimental.pallas import tpu_sc as plsc

N = 10000
E = 320000
D = 128

NC = 2
NS = 16
NW = NC * NS
EPW = E // NW
C = 128
CHUNKS = 3 * (-(-EPW // (3 * C)))
EPW_PAD = CHUNKS * C
N_PAD = 10112
RPT = N_PAD // NS


def _sc_body(xn_hbm, src_hbm, dst_hbm, out_hbm,
             src_v, dst_v, rows_v, rows_v2, rows_v3, tbl_sh, acc_sh,
             gsem, gsem2, gsem3):
    c = lax.axis_index("c")
    s = lax.axis_index("s")
    w = c * NS + s
    pltpu.sync_copy(src_hbm.at[w], src_v)
    pltpu.sync_copy(dst_hbm.at[w], dst_v)
    @pl.when(s < NS - 1)
    def _():
        pltpu.sync_copy(xn_hbm.at[pl.ds(s * RPT, RPT)],
                        tbl_sh.at[pl.ds(s * RPT, RPT)])

    @pl.when(s == NS - 1)
    def _():
        pltpu.sync_copy(xn_hbm.at[pl.ds((NS - 1) * RPT, N - (NS - 1) * RPT)],
                        tbl_sh.at[pl.ds((NS - 1) * RPT, N - (NS - 1) * RPT)])
    zrow = jnp.zeros((32,), jnp.bfloat16)

    def zero_row(j, carry):
        for k in range(D // 32):
            rows_v[j, pl.ds(k * 32, 32)] = zrow
        return carry

    lax.fori_loop(0, C, zero_row, 0)
    for k in range(RPT // C + 1):
        size = C if k < RPT // C else RPT - (RPT // C) * C
        pltpu.sync_copy(rows_v.at[pl.ds(0, size)],
                        acc_sh.at[pl.ds(s * RPT + k * C, size)])
    plsc.subcore_barrier()

    bufs = (rows_v, rows_v2, rows_v3)
    gsems = (gsem, gsem2, gsem3)

    def group_body(i, carry):
        descs = [
            pltpu.async_copy(tbl_sh.at[src_v.at[3 * i + b]],
                             bufs[b], gsems[b])
            for b in range(3)
        ]
        for b in range(3):
            descs[b].wait()
            pltpu.sync_copy(bufs[b], acc_sh.at[dst_v.at[3 * i + b]],
                            add=True)
        return carry

    lax.fori_loop(0, CHUNKS // 3, group_body, 0)
    plsc.subcore_barrier()
    pltpu.sync_copy(acc_sh.at[pl.ds(s * RPT, RPT)],
                    out_hbm.at[c, pl.ds(s * RPT, RPT)])


def _prescale_body(x_ref, norm_ref, o_ref):
    o_ref[...] = (x_ref[...] * norm_ref[...]).astype(jnp.bfloat16)


def _apply_body(p_ref, norm_ref, w_ref, b_ref, o_ref):
    acc = ((p_ref[0].astype(jnp.float32) + p_ref[1].astype(jnp.float32))
           * norm_ref[...])
    h = lax.dot_general(acc, w_ref[...], (((1,), (1,)), ((), ())),
                        preferred_element_type=jnp.float32)
    o_ref[...] = jnp.maximum(h + b_ref[...], 0.0)


_ROWS_BLK = 1000


def kernel(x, norm, edge_index, W, b):
    xn = pl.pallas_call(
        _prescale_body,
        out_shape=jax.ShapeDtypeStruct((N, D), jnp.bfloat16),
        grid=(N // _ROWS_BLK,),
        in_specs=[pl.BlockSpec((_ROWS_BLK, D), lambda i: (i, 0)),
                  pl.BlockSpec((_ROWS_BLK, 1), lambda i: (i, 0))],
        out_specs=pl.BlockSpec((_ROWS_BLK, D), lambda i: (i, 0)),
    )(x, norm)

    src = edge_index[0].reshape(NW, EPW)
    dst = edge_index[1].reshape(NW, EPW)
    pad_n = EPW_PAD - EPW
    src_p = jnp.pad(src, ((0, 0), (0, pad_n))).reshape(NW, CHUNKS, C)
    trash = (N + (jnp.arange(pad_n, dtype=jnp.int32) % (N_PAD - N)))
    dst_p = jnp.concatenate(
        [dst, jnp.broadcast_to(trash, (NW, pad_n))], axis=1
    ).reshape(NW, CHUNKS, C)

    mesh = plsc.VectorSubcoreMesh(core_axis_name="c", subcore_axis_name="s")
    parts = pl.kernel(
        _sc_body,
        out_type=jax.ShapeDtypeStruct((NC, N_PAD, D), jnp.bfloat16),
        mesh=mesh,
        compiler_params=pltpu.CompilerParams(use_tc_tiling_on_sc=False),
        scratch_types=[
            pltpu.VMEM((CHUNKS, C), jnp.int32),
            pltpu.VMEM((CHUNKS, C), jnp.int32),
            pltpu.VMEM((C, D), jnp.bfloat16),
            pltpu.VMEM((C, D), jnp.bfloat16),
            pltpu.VMEM((C, D), jnp.bfloat16),
            pltpu.VMEM_SHARED((N_PAD, D), jnp.bfloat16),
            pltpu.VMEM_SHARED((N_PAD, D), jnp.bfloat16),
            pltpu.SemaphoreType.DMA,
            pltpu.SemaphoreType.DMA,
            pltpu.SemaphoreType.DMA,
        ],
    )(xn, src_p, dst_p)

    b2 = b.reshape(1, D)
    h = pl.pallas_call(
        _apply_body,
        out_shape=jax.ShapeDtypeStruct((N, D), jnp.float32),
        grid=(N // _ROWS_BLK,),
        in_specs=[
            pl.BlockSpec((NC, _ROWS_BLK, D), lambda i: (0, i, 0)),
            pl.BlockSpec((_ROWS_BLK, 1), lambda i: (i, 0)),
            pl.BlockSpec((D, D), lambda i: (0, 0)),
            pl.BlockSpec((1, D), lambda i: (0, 0)),
        ],
        out_specs=pl.BlockSpec((_ROWS_BLK, D), lambda i: (i, 0)),
    )(parts, norm, W, b2)
    return h
